# baseline (speedup 1.0000x reference)
import functools

import jax
import jax.numpy as jnp
from jax import lax
from jax.experimental import pallas as pl
from jax.experimental.pallas import tpu as pltpu
from jax.experimental.pallas import tpu_sc as plsc

_L = 16
_BT = 2048
_BTC = 1024
_T_SC = 2048


def _sc_add(x, pos_emb, t_sc):
    B, T, D = x.shape
    NC, NS = 2, 16
    NW = NC * NS
    TW = t_sc // NW
    CH = 8
    NCH = TW // CH

    mesh = plsc.VectorSubcoreMesh(core_axis_name="c", subcore_axis_name="s")

    @functools.partial(
        pl.kernel,
        out_type=jax.ShapeDtypeStruct((B, t_sc, D), jnp.float32),
        mesh=mesh,
        scratch_types=[
            [pltpu.VMEM((CH, D), jnp.float32) for _ in range(2)],
            [pltpu.VMEM((CH, D), jnp.float32) for _ in range(B)],
            [pltpu.VMEM((CH, D), jnp.float32) for _ in range(B)],
            [pltpu.SemaphoreType.DMA for _ in range(2)],
            [pltpu.SemaphoreType.DMA for _ in range(B)],
            [pltpu.SemaphoreType.DMA for _ in range(B)],
        ],
    )
    def sc_add(x_hbm, pe_hbm, out_hbm, pe_bufs, x_bufs, o_bufs,
               pe_sems, in_sems, out_sems):
        wid = lax.axis_index("s") * NC + lax.axis_index("c")
        t0 = wid * TW

        def pe_issue(c, par):
            pltpu.async_copy(pe_hbm.at[pl.ds(t0 + c * CH, CH), :],
                             pe_bufs[par], pe_sems[par])

        def do_chunk(c, par, first):
            for b in range(B):
                pltpu.async_copy(
                    x_hbm.at[b, pl.ds(t0 + c * CH, CH), :],
                    x_bufs[b], in_sems[b])
            pltpu.make_async_copy(pe_hbm.at[pl.ds(0, CH), :], pe_bufs[par],
                                  pe_sems[par]).wait()

            @pl.when(c + 1 < NCH)
            def _():
                pe_issue(c + 1, 1 - par)

            for b in range(B):
                pltpu.make_async_copy(x_hbm.at[0, pl.ds(0, CH), :],
                                      x_bufs[b], in_sems[b]).wait()

                cond = c > 0 if first else c >= 0

                @pl.when(cond)
                def _():
                    pltpu.make_async_copy(o_bufs[b],
                                          out_hbm.at[0, pl.ds(0, CH), :],
                                          out_sems[b]).wait()

                for r in range(CH):
                    @plsc.parallel_loop(0, D // _L, unroll=8)
                    def _(i):
                        s = pl.ds(i * _L, _L)
                        o_bufs[b][r, s] = x_bufs[b][r, s] + pe_bufs[par][r, s]

                pltpu.async_copy(
                    o_bufs[b],
                    out_hbm.at[b, pl.ds(t0 + c * CH, CH), :],
                    out_sems[b])

        pe_issue(0, 0)

        def c2_body(c2, carry):
            do_chunk(c2 * 2, 0, True)
            do_chunk(c2 * 2 + 1, 1, False)
            return carry

        lax.fori_loop(0, NCH // 2, c2_body, 0)

        for b in range(B):
            pltpu.make_async_copy(o_bufs[b], out_hbm.at[0, pl.ds(0, CH), :],
                                  out_sems[b]).wait()

    return sc_add(x, pos_emb)


def _tc_body(x_ref, pe_ref, o_ref):
    o_ref[...] = x_ref[...] + pe_ref[...]


def _tc_add(x, pos_emb, t_sc):
    B, T, D = x.shape
    off = t_sc // _BT
    return pl.pallas_call(
        _tc_body,
        grid=((T - t_sc) // _BT, B),
        in_specs=[
            pl.BlockSpec((1, _BT, D), lambda t, b: (b, t + off, 0)),
            pl.BlockSpec((_BT, D), lambda t, b: (t + off, 0)),
        ],
        out_specs=pl.BlockSpec((1, _BT, D), lambda t, b: (b, t + off, 0)),
        out_shape=jax.ShapeDtypeStruct((B, T, D), x.dtype),
    )(x, pos_emb)


def _combine_body(sc_ref, tc_hbm, o_ref):
    del tc_hbm
    o_ref[...] = sc_ref[...]


def _combine(sc_out, tc_full, t_sc):
    B, T, D = tc_full.shape
    return pl.pallas_call(
        _combine_body,
        grid=(t_sc // _BTC, B),
        in_specs=[
            pl.BlockSpec((1, _BTC, D), lambda t, b: (b, t, 0)),
            pl.BlockSpec(memory_space=pl.ANY),
        ],
        out_specs=pl.BlockSpec((1, _BTC, D), lambda t, b: (b, t, 0)),
        out_shape=jax.ShapeDtypeStruct((B, T, D), tc_full.dtype),
        input_output_aliases={1: 0},
    )(sc_out, tc_full)


def kernel(x, pos_emb):
    B, T, D = x.shape
    pe = pos_emb[:T]
    sc_out = _sc_add(x, pe, _T_SC)
    tc_full = _tc_add(x, pe, _T_SC)
    return _combine(sc_out, tc_full, _T_SC)


# speedup vs baseline: 1.4326x; 1.4326x over previous
import jax
import jax.numpy as jnp
from jax.experimental import pallas as pl


def _add(x_ref, pe_ref, o_ref):
    o_ref[...] = x_ref[...] + pe_ref[...]


def kernel(x, pos_emb):
    B, T, D = x.shape
    BT = 2048
    return pl.pallas_call(
        _add,
        grid=(T // BT, B),
        in_specs=[
            pl.BlockSpec((1, BT, D), lambda t, b: (b, t, 0)),
            pl.BlockSpec((BT, D), lambda t, b: (t, 0)),
        ],
        out_specs=pl.BlockSpec((1, BT, D), lambda t, b: (b, t, 0)),
        out_shape=jax.ShapeDtypeStruct((B, T, D), x.dtype),
    )(x, pos_emb[:T])
